# baseline (speedup 1.0000x reference)
import functools
import jax
import jax.numpy as jnp
from jax import lax
from jax.experimental import pallas as pl
from jax.experimental.pallas import tpu as pltpu
from jax.experimental.pallas import tpu_sc as plsc

BS = 128
NUM_EXPERTS = 64
TOP_K = 8
PROMPT_LEN = 64
D_MODEL = 1024
PB = 16

NUM_CORES = 2
NUM_SUBCORES = 16
NUM_WORKERS = NUM_CORES * NUM_SUBCORES
TOK_PER_W = BS // NUM_WORKERS
IDX_PER_W = TOK_PER_W * TOP_K
W_PER_W = TOK_PER_W * NUM_EXPERTS


def _w_routing_body(score_hbm, idxf_hbm, eids_hbm, w_hbm,
                    idx_v, score_v, eids_v, w_v):
    wid = lax.axis_index("s") * NUM_CORES + lax.axis_index("c")
    pltpu.sync_copy(idxf_hbm.at[pl.ds(wid * IDX_PER_W, IDX_PER_W)], idx_v)
    pltpu.sync_copy(score_hbm.at[pl.ds(wid * W_PER_W, W_PER_W)], score_v)
    pltpu.sync_copy(eids_hbm, eids_v)

    e_chunks = [eids_v[pl.ds(16 * c, 16)] for c in range(NUM_EXPERTS // 16)]
    idx_vecs = [idx_v[pl.ds(16 * j, 16)] for j in range(IDX_PER_W // 16)]
    one = jnp.full((16,), 1.0, jnp.float32)
    zero = jnp.zeros((16,), jnp.float32)
    for r in range(TOK_PER_W):
        picks = []
        for k in range(TOP_K):
            pos = r * TOP_K + k
            s = idx_vecs[pos // 16][pos % 16]
            picks.append(jnp.full((16,), s, jnp.float32))
        for c in range(NUM_EXPERTS // 16):
            cnt = zero
            for p in picks:
                d = p - e_chunks[c]
                cnt += jnp.maximum(one - d * d, zero)
            off = r * NUM_EXPERTS + 16 * c
            w_v[pl.ds(off, 16)] = cnt * score_v[pl.ds(off, 16)]

    pltpu.sync_copy(w_v, w_hbm.at[pl.ds(wid * W_PER_W, W_PER_W)])


_w_routing = functools.partial(
    pl.kernel,
    mesh=plsc.VectorSubcoreMesh(core_axis_name="c", subcore_axis_name="s"),
    out_type=jax.ShapeDtypeStruct((BS * NUM_EXPERTS,), jnp.float32),
    scratch_types=[
        pltpu.VMEM((IDX_PER_W,), jnp.float32),
        pltpu.VMEM((W_PER_W,), jnp.float32),
        pltpu.VMEM((NUM_EXPERTS,), jnp.float32),
        pltpu.VMEM((W_PER_W,), jnp.float32),
    ],
)(_w_routing_body)


def _moe_kernel(w_in_ref, experts_ref, out_ref):
    w = w_in_ref[...]
    xt = jnp.transpose(experts_ref[...], (1, 0, 2))
    outs = [jnp.dot(w, xt[p], preferred_element_type=jnp.float32)
            for p in range(PB)]
    out_ref[...] = jnp.transpose(jnp.stack(outs, axis=0), (1, 0, 2))


def kernel(selection_score, expert_indices, experts):
    idxf_flat = expert_indices.astype(jnp.float32).reshape(-1)
    score_flat = selection_score.reshape(-1)
    eids = jnp.arange(NUM_EXPERTS, dtype=jnp.float32)
    w = _w_routing(score_flat, idxf_flat, eids).reshape(BS, NUM_EXPERTS)
    out = pl.pallas_call(
        _moe_kernel,
        grid=(PROMPT_LEN // PB,),
        in_specs=[
            pl.BlockSpec((BS, NUM_EXPERTS), lambda j: (0, 0)),
            pl.BlockSpec((NUM_EXPERTS, PB, D_MODEL), lambda j: (0, j, 0)),
        ],
        out_specs=pl.BlockSpec((BS, PB, D_MODEL), lambda j: (0, j, 0)),
        out_shape=jax.ShapeDtypeStruct((BS, PROMPT_LEN, D_MODEL), jnp.float32),
    )(w, experts)
    return out


# speedup vs baseline: 1.0347x; 1.0347x over previous
import functools
import jax
import jax.numpy as jnp
from jax import lax
from jax.experimental import pallas as pl
from jax.experimental.pallas import tpu as pltpu
from jax.experimental.pallas import tpu_sc as plsc

BS = 128
NUM_EXPERTS = 64
TOP_K = 8
PROMPT_LEN = 64
D_MODEL = 1024
PB = 16

NUM_CORES = 2
NUM_SUBCORES = 16
NUM_WORKERS = NUM_CORES * NUM_SUBCORES
TOK_PER_W = BS // NUM_WORKERS
IDX_PER_W = TOK_PER_W * TOP_K
W_PER_W = TOK_PER_W * NUM_EXPERTS


def _w_routing_body(score_hbm, idxf_hbm, eids_hbm, w_hbm,
                    idx_v, score_v, eids_v, w_v):
    wid = lax.axis_index("s") * NUM_CORES + lax.axis_index("c")
    base = wid * TOK_PER_W
    pltpu.sync_copy(idxf_hbm.at[pl.ds(wid * IDX_PER_W, IDX_PER_W)], idx_v)
    pltpu.sync_copy(score_hbm.at[pl.ds(base, TOK_PER_W)], score_v)
    pltpu.sync_copy(eids_hbm, eids_v)

    e_chunks = [eids_v[pl.ds(16 * c, 16)] for c in range(NUM_EXPERTS // 16)]
    idx_vecs = [idx_v[pl.ds(16 * j, 16)] for j in range(IDX_PER_W // 16)]
    one = jnp.full((16,), 1.0, jnp.float32)
    zero = jnp.zeros((16,), jnp.float32)
    for r in range(TOK_PER_W):
        picks = []
        for k in range(TOP_K):
            pos = r * TOP_K + k
            s = idx_vecs[pos // 16][pos % 16]
            picks.append(jnp.full((16,), s, jnp.float32))
        for c in range(NUM_EXPERTS // 16):
            cnt = zero
            for p in picks:
                d = p - e_chunks[c]
                cnt += jnp.maximum(one - d * d, zero)
            off = 16 * c
            w_v[r, pl.ds(off, 16)] = cnt * score_v[r, pl.ds(off, 16)]

    pltpu.sync_copy(w_v, w_hbm.at[pl.ds(base, TOK_PER_W)])


_w_routing = functools.partial(
    pl.kernel,
    mesh=plsc.VectorSubcoreMesh(core_axis_name="c", subcore_axis_name="s"),
    out_type=jax.ShapeDtypeStruct((BS, NUM_EXPERTS), jnp.float32),
    scratch_types=[
        pltpu.VMEM((IDX_PER_W,), jnp.float32),
        pltpu.VMEM((TOK_PER_W, NUM_EXPERTS), jnp.float32),
        pltpu.VMEM((NUM_EXPERTS,), jnp.float32),
        pltpu.VMEM((TOK_PER_W, NUM_EXPERTS), jnp.float32),
    ],
)(_w_routing_body)


def _moe_kernel(w_in_ref, experts_ref, out_ref):
    w = w_in_ref[...]
    xt = jnp.transpose(experts_ref[...], (1, 0, 2))
    outs = [jnp.dot(w, xt[p], preferred_element_type=jnp.float32)
            for p in range(PB)]
    out_ref[...] = jnp.transpose(jnp.stack(outs, axis=0), (1, 0, 2))


def kernel(selection_score, expert_indices, experts):
    idxf_flat = expert_indices.astype(jnp.float32).reshape(-1)
    eids = jnp.arange(NUM_EXPERTS, dtype=jnp.float32)
    w = _w_routing(selection_score, idxf_flat, eids)
    out = pl.pallas_call(
        _moe_kernel,
        grid=(PROMPT_LEN // PB,),
        in_specs=[
            pl.BlockSpec((BS, NUM_EXPERTS), lambda j: (0, 0)),
            pl.BlockSpec((NUM_EXPERTS, PB, D_MODEL), lambda j: (0, j, 0)),
        ],
        out_specs=pl.BlockSpec((BS, PB, D_MODEL), lambda j: (0, j, 0)),
        out_shape=jax.ShapeDtypeStruct((BS, PROMPT_LEN, D_MODEL), jnp.float32),
    )(w, experts)
    return out


# speedup vs baseline: 1.0405x; 1.0056x over previous
import functools
import jax
import jax.numpy as jnp
from jax import lax
from jax.experimental import pallas as pl
from jax.experimental.pallas import tpu as pltpu
from jax.experimental.pallas import tpu_sc as plsc

BS = 128
NUM_EXPERTS = 64
TOP_K = 8
PROMPT_LEN = 64
D_MODEL = 1024
PB = 16

NUM_CORES = 2
NUM_SUBCORES = 16
NUM_WORKERS = NUM_CORES * NUM_SUBCORES
TOK_PER_W = BS // NUM_WORKERS
IDX_PER_W = TOK_PER_W * TOP_K


def _w_routing_body(score_hbm, idxf_hbm, eids_hbm, w_hbm,
                    idx_v, score_v, eids_v, w_v):
    wid = lax.axis_index("s") * NUM_CORES + lax.axis_index("c")
    base = wid * TOK_PER_W
    pltpu.sync_copy(idxf_hbm.at[pl.ds(wid * IDX_PER_W, IDX_PER_W)], idx_v)
    pltpu.sync_copy(score_hbm.at[pl.ds(base, TOK_PER_W)], score_v)
    pltpu.sync_copy(eids_hbm, eids_v)

    e_chunks = [eids_v[pl.ds(16 * c, 16)] for c in range(NUM_EXPERTS // 16)]
    idx_vecs = [idx_v[pl.ds(16 * j, 16)] for j in range(IDX_PER_W // 16)]
    one = jnp.full((16,), 1.0, jnp.float32)
    zero = jnp.zeros((16,), jnp.float32)
    for r in range(TOK_PER_W):
        picks = []
        for k in range(TOP_K):
            pos = r * TOP_K + k
            s = idx_vecs[pos // 16][pos % 16]
            picks.append(jnp.full((16,), s, jnp.float32))
        for c in range(NUM_EXPERTS // 16):
            cnt = zero
            for p in picks:
                d = p - e_chunks[c]
                cnt += jnp.maximum(one - d * d, zero)
            off = 16 * c
            w_v[r, pl.ds(off, 16)] = cnt * score_v[r, pl.ds(off, 16)]

    pltpu.sync_copy(w_v, w_hbm.at[pl.ds(base, TOK_PER_W)])


_w_routing = functools.partial(
    pl.kernel,
    mesh=plsc.VectorSubcoreMesh(core_axis_name="c", subcore_axis_name="s"),
    out_type=jax.ShapeDtypeStruct((BS, NUM_EXPERTS), jnp.float32),
    scratch_types=[
        pltpu.VMEM((IDX_PER_W,), jnp.float32),
        pltpu.VMEM((TOK_PER_W, NUM_EXPERTS), jnp.float32),
        pltpu.VMEM((NUM_EXPERTS,), jnp.float32),
        pltpu.VMEM((TOK_PER_W, NUM_EXPERTS), jnp.float32),
    ],
)(_w_routing_body)


def _moe_kernel(w_in_ref, experts_ref, out_ref):
    w = w_in_ref[...]
    xt = jnp.transpose(experts_ref[...], (1, 0, 2))
    outs = [jnp.dot(w, xt[p], preferred_element_type=jnp.float32)
            for p in range(PB)]
    out_ref[...] = jnp.transpose(jnp.stack(outs, axis=0), (1, 0, 2))


def kernel(selection_score, expert_indices, experts):
    idxf_flat = expert_indices.astype(jnp.float32).reshape(-1)
    eids = jnp.arange(NUM_EXPERTS, dtype=jnp.float32)
    w = _w_routing(selection_score, idxf_flat, eids)
    out = pl.pallas_call(
        _moe_kernel,
        grid=(PROMPT_LEN // PB,),
        in_specs=[
            pl.BlockSpec((BS, NUM_EXPERTS), lambda j: (0, 0)),
            pl.BlockSpec((NUM_EXPERTS, PB, D_MODEL), lambda j: (0, j, 0)),
        ],
        out_specs=pl.BlockSpec((BS, PB, D_MODEL), lambda j: (0, j, 0)),
        out_shape=jax.ShapeDtypeStruct((BS, PROMPT_LEN, D_MODEL), jnp.float32),
    )(w, experts)
    return out
